# Initial kernel scaffold; baseline (speedup 1.0000x reference)
import jax
import jax.numpy as jnp
from jax.experimental import pallas as pl


def kernel(x, length, embed_table, lin_w, lin_b):
    raise NotImplementedError("write your pallas kernel here")


# speedup vs baseline: 171.9500x; 171.9500x over previous
import functools

import jax
import jax.numpy as jnp
from jax import lax
from jax.experimental import pallas as pl
from jax.experimental.pallas import tpu as pltpu
from jax.experimental.pallas import tpu_sc as plsc

_B = 16384
_L = 200
_VOCAB = 7800
_DIM = 64

_NC = 2
_NS = 16
_NW = _NC * _NS
_LANES = 16
_ROWS_PER_W = _B // _NW
_GROUPS = _ROWS_PER_W // _LANES
_UNROLL = 8


def _scores_body(table_ref, w_ref, s_ref):
    s_ref[...] = jnp.sum(table_ref[...] * w_ref[...], axis=1)


def _vocab_scores(embed_table, lin_w):
    return pl.pallas_call(
        _scores_body,
        out_shape=jax.ShapeDtypeStruct((_VOCAB,), jnp.float32),
    )(embed_table, lin_w)


def _sc_pool(x_flat, length, scores, bias16):
    mesh = plsc.VectorSubcoreMesh(core_axis_name="c", subcore_axis_name="s")

    @functools.partial(
        pl.kernel,
        mesh=mesh,
        compiler_params=pltpu.CompilerParams(needs_layout_passes=False),
        out_type=jax.ShapeDtypeStruct((_B,), jnp.float32),
        scratch_types=[
            pltpu.VMEM((_VOCAB,), jnp.float32),
            pltpu.VMEM((_ROWS_PER_W * _L,), jnp.int32),
            pltpu.VMEM((_ROWS_PER_W,), jnp.float32),
            pltpu.VMEM((_LANES,), jnp.float32),
            pltpu.VMEM((_ROWS_PER_W,), jnp.float32),
        ],
    )
    def pool(x_hbm, len_hbm, s_hbm, b_hbm, out_hbm, s_v, x_v, len_v, b_v, out_v):
        wid = lax.axis_index("s") * _NC + lax.axis_index("c")
        row0 = wid * _ROWS_PER_W
        pltpu.sync_copy(s_hbm, s_v)
        pltpu.sync_copy(x_hbm.at[pl.ds(row0 * _L, _ROWS_PER_W * _L)], x_v)
        pltpu.sync_copy(len_hbm.at[pl.ds(row0, _ROWS_PER_W)], len_v)
        pltpu.sync_copy(b_hbm, b_v)
        lane_row = lax.iota(jnp.int32, _LANES) * _L
        bias = b_v[...]

        def group(g, carry):
            pos0 = lane_row + g * (_LANES * _L)

            def step(i, acc):
                a = acc
                for u in range(_UNROLL):
                    pos = pos0 + (i * _UNROLL + u)
                    xi = plsc.load_gather(x_v, [pos])
                    a = a + plsc.load_gather(s_v, [xi])
                return a

            acc = lax.fori_loop(0, _L // _UNROLL, step,
                                jnp.zeros((_LANES,), jnp.float32))
            t = acc / len_v[pl.ds(g * _LANES, _LANES)] + bias
            out_v[pl.ds(g * _LANES, _LANES)] = 1.0 / (1.0 + jnp.exp(-t))
            return carry

        lax.fori_loop(0, _GROUPS, group, 0)
        pltpu.sync_copy(out_v, out_hbm.at[pl.ds(row0, _ROWS_PER_W)])

    return pool(x_flat, length, scores, bias16)


@jax.jit
def kernel(x, length, embed_table, lin_w, lin_b):
    scores = _vocab_scores(embed_table, lin_w)
    bias16 = jnp.broadcast_to(lin_b.astype(jnp.float32), (_LANES,))
    y = _sc_pool(x.reshape(-1), length, scores, bias16)
    return y.reshape(_B, 1)
